# Initial kernel scaffold; baseline (speedup 1.0000x reference)
import jax
import jax.numpy as jnp
from jax.experimental import pallas as pl


def kernel(x, edge_index, W, b):
    raise NotImplementedError("write your pallas kernel here")


# speedup vs baseline: 13.8142x; 13.8142x over previous
import functools

import jax
import jax.numpy as jnp
from jax import lax
from jax.experimental import pallas as pl
from jax.experimental.pallas import tpu as pltpu
from jax.experimental.pallas import tpu_sc as plsc

NC = 2
NS = 16
NW = NC * NS
CHUNK = 128


def _sc_mesh():
    return plsc.VectorSubcoreMesh(
        core_axis_name="c", subcore_axis_name="s", num_cores=NC, num_subcores=NS
    )


def _k1_body(n_pad, ept, nchunk, dst_hbm, out_hbm, idx_v, ones_v, tmp_v, deg_sp):
    c = lax.axis_index("c")
    s = lax.axis_index("s")
    base = (c * NS + s) * ept
    rows_per = n_pad // NS

    for j in range(CHUNK // 16):
        ones_v[pl.ds(j * 16, 16)] = jnp.ones((16,), jnp.float32)

    def zbody(j, carry):
        tmp_v[pl.ds(j * 16, 16)] = jnp.zeros((16,), jnp.float32)
        return carry

    lax.fori_loop(0, rows_per // 16, zbody, 0)
    pltpu.sync_copy(tmp_v, deg_sp.at[pl.ds(s * rows_per, rows_per)])
    plsc.subcore_barrier()

    def ebody(ci, carry):
        pltpu.sync_copy(dst_hbm.at[pl.ds(base + ci * CHUNK, CHUNK)], idx_v)
        pltpu.sync_copy(ones_v, deg_sp.at[idx_v], add=True)
        return carry

    lax.fori_loop(0, nchunk, ebody, 0)
    plsc.subcore_barrier()

    pltpu.sync_copy(deg_sp.at[pl.ds(s * rows_per, rows_per)], tmp_v)
    pltpu.sync_copy(tmp_v, out_hbm.at[pl.ds(c * n_pad + s * rows_per, rows_per)])


def _deg_partials(dst_pad, n_pad):
    e_pad = dst_pad.shape[0]
    ept = e_pad // NW
    nchunk = ept // CHUNK
    rows_per = n_pad // NS
    k = pl.kernel(
        functools.partial(_k1_body, n_pad, ept, nchunk),
        out_type=jax.ShapeDtypeStruct((NC * n_pad,), jnp.float32),
        mesh=_sc_mesh(),
        scratch_types=[
            pltpu.VMEM((CHUNK,), jnp.int32),
            pltpu.VMEM((CHUNK,), jnp.float32),
            pltpu.VMEM((rows_per,), jnp.float32),
            pltpu.VMEM_SHARED((n_pad,), jnp.float32),
        ],
    )
    return k(dst_pad)


def _k2_body(x_ref, w_ref, d0_ref, d1_ref, g_ref):
    deg = d0_ref[...] + d1_ref[...] + 1.0
    dis = lax.rsqrt(deg)
    h = jnp.dot(x_ref[...], w_ref[...], preferred_element_type=jnp.float32)
    g_ref[...] = h * dis


def _scaled_transform(x_pad, w, deg0, deg1):
    n_pad, d_in = x_pad.shape
    d_out = w.shape[1]
    blk = 512
    grid = (n_pad // blk,)
    return pl.pallas_call(
        _k2_body,
        grid=grid,
        in_specs=[
            pl.BlockSpec((blk, d_in), lambda i: (i, 0)),
            pl.BlockSpec((d_in, d_out), lambda i: (0, 0)),
            pl.BlockSpec((blk, 1), lambda i: (i, 0)),
            pl.BlockSpec((blk, 1), lambda i: (i, 0)),
        ],
        out_specs=pl.BlockSpec((blk, d_out), lambda i: (i, 0)),
        out_shape=jax.ShapeDtypeStruct((n_pad, d_out), jnp.float32),
    )(x_pad, w, deg0, deg1)


def _k3_body(n_pad, ept, nchunk, g_hbm, src_hbm, dst_hbm, out_hbm,
             idx_s, idx_d, rows_v, acc_sp):
    c = lax.axis_index("c")
    s = lax.axis_index("s")
    base = (c * NS + s) * ept
    rows_per = n_pad // NS

    def zrow(r, carry):
        def zcol(j, carry2):
            rows_v[r, pl.ds(j * 16, 16)] = jnp.zeros((16,), jnp.float32)
            return carry2
        return lax.fori_loop(0, 8, zcol, carry)

    lax.fori_loop(0, CHUNK, zrow, 0)

    def ibody(k_, carry):
        pltpu.sync_copy(rows_v, acc_sp.at[pl.ds(s * rows_per + k_ * CHUNK, CHUNK)])
        return carry

    lax.fori_loop(0, rows_per // CHUNK, ibody, 0)
    plsc.subcore_barrier()

    def ebody(ci, carry):
        off = base + ci * CHUNK
        pltpu.sync_copy(src_hbm.at[pl.ds(off, CHUNK)], idx_s)
        pltpu.sync_copy(dst_hbm.at[pl.ds(off, CHUNK)], idx_d)
        pltpu.sync_copy(g_hbm.at[idx_s], rows_v)
        pltpu.sync_copy(rows_v, acc_sp.at[idx_d], add=True)
        return carry

    lax.fori_loop(0, nchunk, ebody, 0)
    plsc.subcore_barrier()

    def obody(k_, carry):
        r0 = s * rows_per + k_ * CHUNK
        pltpu.sync_copy(acc_sp.at[pl.ds(r0, CHUNK)], rows_v)
        pltpu.sync_copy(rows_v, out_hbm.at[pl.ds(c * n_pad + r0, CHUNK)])
        return carry

    lax.fori_loop(0, rows_per // CHUNK, obody, 0)


def _aggregate(g, src_pad, dst_pad):
    n_pad, d = g.shape
    e_pad = src_pad.shape[0]
    ept = e_pad // NW
    nchunk = ept // CHUNK
    k = pl.kernel(
        functools.partial(_k3_body, n_pad, ept, nchunk),
        out_type=jax.ShapeDtypeStruct((NC * n_pad, d), jnp.float32),
        mesh=_sc_mesh(),
        scratch_types=[
            pltpu.VMEM((CHUNK,), jnp.int32),
            pltpu.VMEM((CHUNK,), jnp.int32),
            pltpu.VMEM((CHUNK, d), jnp.float32),
            pltpu.VMEM_SHARED((n_pad, d), jnp.float32),
        ],
    )
    return k(g, src_pad, dst_pad)


def _k4_body(x_ref, a0_ref, a1_ref, g_ref, d0_ref, d1_ref, b_ref, o_ref):
    dis = lax.rsqrt(d0_ref[...] + d1_ref[...] + 1.0)
    t = dis * (a0_ref[...] + a1_ref[...] + g_ref[...]) + b_ref[...]
    o_ref[...] = x_ref[...] + jnp.maximum(t, 0.0)


def _epilogue(x, a0, a1, g, deg0, deg1, b2d):
    n, d = x.shape
    blk = 2000
    grid = (n // blk,)
    row_spec = pl.BlockSpec((blk, d), lambda i: (i, 0))
    col_spec = pl.BlockSpec((blk, 1), lambda i: (i, 0))
    return pl.pallas_call(
        _k4_body,
        grid=grid,
        in_specs=[row_spec, row_spec, row_spec, row_spec, col_spec, col_spec,
                  pl.BlockSpec((1, d), lambda i: (0, 0))],
        out_specs=row_spec,
        out_shape=jax.ShapeDtypeStruct((n, d), jnp.float32),
    )(x, a0, a1, g, deg0, deg1, b2d)


def kernel(x, edge_index, W, b):
    n, d_in = x.shape
    d_out = W.shape[1]
    e = edge_index.shape[1]

    n_pad = ((n + (NS * CHUNK) - 1) // (NS * CHUNK)) * (NS * CHUNK)
    ept = ((e + NW - 1) // NW + CHUNK - 1) // CHUNK * CHUNK
    e_pad = ept * NW

    src = edge_index[0].astype(jnp.int32)
    dst = edge_index[1].astype(jnp.int32)
    pad_idx = jnp.full((e_pad - e,), n, dtype=jnp.int32)
    src_pad = jnp.concatenate([src, pad_idx])
    dst_pad = jnp.concatenate([dst, pad_idx])

    x_pad = jnp.pad(x, ((0, n_pad - n), (0, 0)))

    degp = _deg_partials(dst_pad, n_pad)
    deg0 = degp[:n_pad].reshape(n_pad, 1)
    deg1 = degp[n_pad:].reshape(n_pad, 1)

    g = _scaled_transform(x_pad, W, deg0, deg1)

    acc = _aggregate(g, src_pad, dst_pad)

    out = _epilogue(
        x, acc[:n], acc[n_pad:n_pad + n], g[:n],
        deg0[:n], deg1[:n], b.reshape(1, d_out)
    )
    return out


# speedup vs baseline: 14.2716x; 1.0331x over previous
import functools

import jax
import jax.numpy as jnp
from jax import lax
from jax.experimental import pallas as pl
from jax.experimental.pallas import tpu as pltpu
from jax.experimental.pallas import tpu_sc as plsc

NC = 2
NS = 16
NW = NC * NS
CHUNK = 128


def _sc_mesh():
    return plsc.VectorSubcoreMesh(
        core_axis_name="c", subcore_axis_name="s", num_cores=NC, num_subcores=NS
    )


def _k1_body(n_pad, nchunk, dst2d_hbm, out_hbm, idx_all, ones_v, tmp_v, deg_sp):
    c = lax.axis_index("c")
    s = lax.axis_index("s")
    wid = c * NS + s
    rows_per = n_pad // NS

    pltpu.sync_copy(dst2d_hbm.at[pl.ds(wid * nchunk, nchunk)], idx_all)

    for j in range(CHUNK // 16):
        ones_v[pl.ds(j * 16, 16)] = jnp.ones((16,), jnp.float32)

    def zbody(j, carry):
        tmp_v[pl.ds(j * 16, 16)] = jnp.zeros((16,), jnp.float32)
        return carry

    lax.fori_loop(0, rows_per // 16, zbody, 0)
    pltpu.sync_copy(tmp_v, deg_sp.at[pl.ds(s * rows_per, rows_per)])
    plsc.subcore_barrier()

    def ebody(ci, carry):
        pltpu.sync_copy(ones_v, deg_sp.at[idx_all.at[ci]], add=True)
        return carry

    lax.fori_loop(0, nchunk, ebody, 0)
    plsc.subcore_barrier()

    pltpu.sync_copy(deg_sp.at[pl.ds(s * rows_per, rows_per)], tmp_v)
    pltpu.sync_copy(tmp_v, out_hbm.at[pl.ds(c * n_pad + s * rows_per, rows_per)])


def _deg_partials(dst2d, n_pad):
    nrow = dst2d.shape[0]
    nchunk = nrow // NW
    rows_per = n_pad // NS
    k = pl.kernel(
        functools.partial(_k1_body, n_pad, nchunk),
        out_type=jax.ShapeDtypeStruct((NC * n_pad,), jnp.float32),
        mesh=_sc_mesh(),
        scratch_types=[
            pltpu.VMEM((nchunk, CHUNK), jnp.int32),
            pltpu.VMEM((CHUNK,), jnp.float32),
            pltpu.VMEM((rows_per,), jnp.float32),
            pltpu.VMEM_SHARED((n_pad,), jnp.float32),
        ],
    )
    return k(dst2d)


def _k2_body(x_ref, w_ref, d0_ref, d1_ref, g_ref):
    deg = d0_ref[...] + d1_ref[...] + 1.0
    dis = lax.rsqrt(deg)
    h = jnp.dot(x_ref[...], w_ref[...], preferred_element_type=jnp.float32)
    g_ref[...] = h * dis


def _scaled_transform(x_pad, w, deg0, deg1):
    n_pad, d_in = x_pad.shape
    d_out = w.shape[1]
    blk = 512
    grid = (n_pad // blk,)
    return pl.pallas_call(
        _k2_body,
        grid=grid,
        in_specs=[
            pl.BlockSpec((blk, d_in), lambda i: (i, 0)),
            pl.BlockSpec((d_in, d_out), lambda i: (0, 0)),
            pl.BlockSpec((blk, 1), lambda i: (i, 0)),
            pl.BlockSpec((blk, 1), lambda i: (i, 0)),
        ],
        out_specs=pl.BlockSpec((blk, d_out), lambda i: (i, 0)),
        out_shape=jax.ShapeDtypeStruct((n_pad, d_out), jnp.float32),
    )(x_pad, w, deg0, deg1)


def _k3_body(n_pad, nchunk, g_hbm, src2d_hbm, dst2d_hbm, out_hbm,
             idx_s, idx_d, rows0, rows1, acc_sp, sem0, sem1, semd0, semd1):
    c = lax.axis_index("c")
    s = lax.axis_index("s")
    wid = c * NS + s
    rows_per = n_pad // NS

    pltpu.sync_copy(src2d_hbm.at[pl.ds(wid * nchunk, nchunk)], idx_s)

    def zrow(r, carry):
        def zcol(j, carry2):
            rows0[r, pl.ds(j * 16, 16)] = jnp.zeros((16,), jnp.float32)
            return carry2
        return lax.fori_loop(0, 8, zcol, carry)

    lax.fori_loop(0, CHUNK, zrow, 0)

    def ibody(k_, carry):
        pltpu.sync_copy(rows0, acc_sp.at[pl.ds(s * rows_per + k_ * CHUNK, CHUNK)])
        return carry

    lax.fori_loop(0, rows_per // CHUNK, ibody, 0)
    plsc.subcore_barrier()

    def gather(j, rows, sem):
        pltpu.async_copy(g_hbm.at[idx_s.at[j]], rows, sem)

    def gwait(rows, sem):
        pltpu.make_async_copy(g_hbm.at[idx_s.at[0]], rows, sem).wait()

    dbase = wid * nchunk

    def dload(j, k, sem):
        pltpu.async_copy(dst2d_hbm.at[dbase + j], idx_d.at[k], sem)

    def dwait(k, sem):
        pltpu.make_async_copy(dst2d_hbm.at[dbase], idx_d.at[k], sem).wait()

    dload(0, 0, semd0)
    gather(0, rows0, sem0)

    def pair(p, carry):
        j0 = p * 2
        j1 = j0 + 1
        dload(j1, 1, semd1)
        gwait(rows0, sem0)
        gather(j1, rows1, sem1)
        dwait(0, semd0)
        pltpu.sync_copy(rows0, acc_sp.at[idx_d.at[0]], add=True)
        gwait(rows1, sem1)

        @pl.when(j1 + 1 < nchunk)
        def _():
            dload(j1 + 1, 0, semd0)
            gather(j1 + 1, rows0, sem0)

        dwait(1, semd1)
        pltpu.sync_copy(rows1, acc_sp.at[idx_d.at[1]], add=True)
        return carry

    lax.fori_loop(0, nchunk // 2, pair, 0)
    plsc.subcore_barrier()

    def obody(k_, carry):
        r0 = s * rows_per + k_ * CHUNK
        pltpu.sync_copy(acc_sp.at[pl.ds(r0, CHUNK)], rows0)
        pltpu.sync_copy(rows0, out_hbm.at[pl.ds(c * n_pad + r0, CHUNK)])
        return carry

    lax.fori_loop(0, rows_per // CHUNK, obody, 0)


def _aggregate(g, src2d, dst2d):
    n_pad, d = g.shape
    nrow = src2d.shape[0]
    nchunk = nrow // NW
    k = pl.kernel(
        functools.partial(_k3_body, n_pad, nchunk),
        out_type=jax.ShapeDtypeStruct((NC * n_pad, d), jnp.float32),
        mesh=_sc_mesh(),
        scratch_types=[
            pltpu.VMEM((nchunk, CHUNK), jnp.int32),
            pltpu.VMEM((2, CHUNK), jnp.int32),
            pltpu.VMEM((CHUNK, d), jnp.float32),
            pltpu.VMEM((CHUNK, d), jnp.float32),
            pltpu.VMEM_SHARED((n_pad, d), jnp.float32),
            pltpu.SemaphoreType.DMA,
            pltpu.SemaphoreType.DMA,
            pltpu.SemaphoreType.DMA,
            pltpu.SemaphoreType.DMA,
        ],
    )
    return k(g, src2d, dst2d)


def _k4_body(x_ref, a0_ref, a1_ref, g_ref, d0_ref, d1_ref, b_ref, o_ref):
    dis = lax.rsqrt(d0_ref[...] + d1_ref[...] + 1.0)
    t = dis * (a0_ref[...] + a1_ref[...] + g_ref[...]) + b_ref[...]
    o_ref[...] = x_ref[...] + jnp.maximum(t, 0.0)


def _epilogue(x, a0, a1, g, deg0, deg1, b2d):
    n, d = x.shape
    blk = 2000
    grid = (n // blk,)
    row_spec = pl.BlockSpec((blk, d), lambda i: (i, 0))
    col_spec = pl.BlockSpec((blk, 1), lambda i: (i, 0))
    return pl.pallas_call(
        _k4_body,
        grid=grid,
        in_specs=[row_spec, row_spec, row_spec, row_spec, col_spec, col_spec,
                  pl.BlockSpec((1, d), lambda i: (0, 0))],
        out_specs=row_spec,
        out_shape=jax.ShapeDtypeStruct((n, d), jnp.float32),
    )(x, a0, a1, g, deg0, deg1, b2d)


def kernel(x, edge_index, W, b):
    n, d_in = x.shape
    d_out = W.shape[1]
    e = edge_index.shape[1]

    n_pad = ((n + (NS * CHUNK) - 1) // (NS * CHUNK)) * (NS * CHUNK)
    ept = ((e + NW - 1) // NW + 2 * CHUNK - 1) // (2 * CHUNK) * (2 * CHUNK)
    e_pad = ept * NW

    src = edge_index[0].astype(jnp.int32)
    dst = edge_index[1].astype(jnp.int32)
    pad_idx = jnp.full((e_pad - e,), n, dtype=jnp.int32)
    src2d = jnp.concatenate([src, pad_idx]).reshape(e_pad // CHUNK, CHUNK)
    dst2d = jnp.concatenate([dst, pad_idx]).reshape(e_pad // CHUNK, CHUNK)

    x_pad = jnp.pad(x, ((0, n_pad - n), (0, 0)))

    degp = _deg_partials(dst2d, n_pad)
    deg0 = degp[:n_pad].reshape(n_pad, 1)
    deg1 = degp[n_pad:].reshape(n_pad, 1)

    g = _scaled_transform(x_pad, W, deg0, deg1)

    acc = _aggregate(g, src2d, dst2d)

    out = _epilogue(
        x, acc[:n], acc[n_pad:n_pad + n], g[:n],
        deg0[:n], deg1[:n], b.reshape(1, d_out)
    )
    return out


# speedup vs baseline: 34.5104x; 2.4181x over previous
import functools

import jax
import jax.numpy as jnp
from jax import lax
from jax.experimental import pallas as pl
from jax.experimental.pallas import tpu as pltpu
from jax.experimental.pallas import tpu_sc as plsc

NC = 2
NS = 16
NW = NC * NS
CHUNK = 128


def _sc_mesh():
    return plsc.VectorSubcoreMesh(
        core_axis_name="c", subcore_axis_name="s", num_cores=NC, num_subcores=NS
    )


def _k1_body(n_pad, ept, dst_hbm, out_hbm, idx_d, idx_t, ones_v, tmp_v, deg_sp,
             semd0, semd1):
    c = lax.axis_index("c")
    s = lax.axis_index("s")
    base = (c * NS + s) * ept
    rows_per = n_pad // NS
    nfull = ept // CHUNK
    tail = ept % CHUNK

    for j in range(CHUNK // 16):
        ones_v[pl.ds(j * 16, 16)] = jnp.ones((16,), jnp.float32)

    def zbody(j, carry):
        tmp_v[pl.ds(j * 16, 16)] = jnp.zeros((16,), jnp.float32)
        return carry

    lax.fori_loop(0, rows_per // 16, zbody, 0)
    pltpu.sync_copy(tmp_v, deg_sp.at[pl.ds(s * rows_per, rows_per)])
    plsc.subcore_barrier()

    def dload(j, k, sem):
        pltpu.async_copy(dst_hbm.at[pl.ds(base + j * CHUNK, CHUNK)],
                         idx_d.at[k], sem)

    def dwait(k, sem):
        pltpu.make_async_copy(dst_hbm.at[pl.ds(0, CHUNK)], idx_d.at[k],
                              sem).wait()

    if nfull > 0:
        dload(0, 0, semd0)

        def pair(p, carry):
            j0 = p * 2
            j1 = j0 + 1

            @pl.when(j1 < nfull)
            def _():
                dload(j1, 1, semd1)

            dwait(0, semd0)
            pltpu.sync_copy(ones_v, deg_sp.at[idx_d.at[0]], add=True)

            @pl.when(j1 < nfull)
            def _():
                @pl.when(j1 + 1 < nfull)
                def _():
                    dload(j1 + 1, 0, semd0)

                dwait(1, semd1)
                pltpu.sync_copy(ones_v, deg_sp.at[idx_d.at[1]], add=True)

            return carry

        lax.fori_loop(0, (nfull + 1) // 2, pair, 0)

    if tail > 0:
        pltpu.sync_copy(dst_hbm.at[pl.ds(base + nfull * CHUNK, tail)], idx_t)
        pltpu.sync_copy(ones_v.at[pl.ds(0, tail)], deg_sp.at[idx_t], add=True)

    plsc.subcore_barrier()
    pltpu.sync_copy(deg_sp.at[pl.ds(s * rows_per, rows_per)], tmp_v)
    pltpu.sync_copy(tmp_v, out_hbm.at[pl.ds(c * n_pad + s * rows_per, rows_per)])


def _deg_partials(dst, n_pad):
    ept = dst.shape[0] // NW
    tail = ept % CHUNK
    rows_per = n_pad // NS
    k = pl.kernel(
        functools.partial(_k1_body, n_pad, ept),
        out_type=jax.ShapeDtypeStruct((NC * n_pad,), jnp.float32),
        mesh=_sc_mesh(),
        scratch_types=[
            pltpu.VMEM((2, CHUNK), jnp.int32),
            pltpu.VMEM((max(tail, 8),), jnp.int32),
            pltpu.VMEM((CHUNK,), jnp.float32),
            pltpu.VMEM((rows_per,), jnp.float32),
            pltpu.VMEM_SHARED((n_pad,), jnp.float32),
            pltpu.SemaphoreType.DMA,
            pltpu.SemaphoreType.DMA,
        ],
    )
    return k(dst)


def _k2_body(x_ref, w_ref, d0_ref, d1_ref, g_ref):
    deg = d0_ref[...] + d1_ref[...] + 1.0
    dis = lax.rsqrt(deg)
    h = jnp.dot(x_ref[...], w_ref[...], preferred_element_type=jnp.float32)
    g_ref[...] = h * dis


def _scaled_transform(x, w, deg0, deg1):
    n, d_in = x.shape
    d_out = w.shape[1]
    blk = 2000
    grid = (n // blk,)
    return pl.pallas_call(
        _k2_body,
        grid=grid,
        in_specs=[
            pl.BlockSpec((blk, d_in), lambda i: (i, 0)),
            pl.BlockSpec((d_in, d_out), lambda i: (0, 0)),
            pl.BlockSpec((blk, 1), lambda i: (i, 0)),
            pl.BlockSpec((blk, 1), lambda i: (i, 0)),
        ],
        out_specs=pl.BlockSpec((blk, d_out), lambda i: (i, 0)),
        out_shape=jax.ShapeDtypeStruct((n, d_out), jnp.float32),
    )(x, w, deg0, deg1)


def _k3_body(n_pad, ept, d, g_hbm, src_hbm, dst_hbm, out_hbm,
             idx_s, idx_d, idx_t, rows0, rows1, acc_sp,
             sem0, sem1, semd0, semd1):
    c = lax.axis_index("c")
    s = lax.axis_index("s")
    base = (c * NS + s) * ept
    rows_per = n_pad // NS
    nfull = ept // CHUNK
    tail = ept % CHUNK

    pltpu.sync_copy(src_hbm.at[pl.ds(base, ept)], idx_s)

    def zrow(r, carry):
        def zcol(j, carry2):
            rows0[r, pl.ds(j * 16, 16)] = jnp.zeros((16,), jnp.float32)
            return carry2
        return lax.fori_loop(0, d // 16, zcol, carry)

    lax.fori_loop(0, CHUNK, zrow, 0)

    def ibody(k_, carry):
        pltpu.sync_copy(rows0, acc_sp.at[pl.ds(s * rows_per + k_ * CHUNK, CHUNK)])
        return carry

    lax.fori_loop(0, rows_per // CHUNK, ibody, 0)
    plsc.subcore_barrier()

    def gather(j, rows, sem):
        pltpu.async_copy(g_hbm.at[idx_s.at[pl.ds(j * CHUNK, CHUNK)]], rows, sem)

    def gwait(rows, sem):
        pltpu.make_async_copy(g_hbm.at[idx_s.at[pl.ds(0, CHUNK)]], rows,
                              sem).wait()

    def dload(j, k, sem):
        pltpu.async_copy(dst_hbm.at[pl.ds(base + j * CHUNK, CHUNK)],
                         idx_d.at[k], sem)

    def dwait(k, sem):
        pltpu.make_async_copy(dst_hbm.at[pl.ds(0, CHUNK)], idx_d.at[k],
                              sem).wait()

    if nfull > 0:
        dload(0, 0, semd0)
        gather(0, rows0, sem0)

        def pair(p, carry):
            j0 = p * 2
            j1 = j0 + 1

            @pl.when(j1 < nfull)
            def _():
                dload(j1, 1, semd1)
                gather(j1, rows1, sem1)

            gwait(rows0, sem0)
            dwait(0, semd0)
            pltpu.sync_copy(rows0, acc_sp.at[idx_d.at[0]], add=True)

            @pl.when(j1 < nfull)
            def _():
                @pl.when(j1 + 1 < nfull)
                def _():
                    dload(j1 + 1, 0, semd0)
                    gather(j1 + 1, rows0, sem0)

                gwait(rows1, sem1)
                dwait(1, semd1)
                pltpu.sync_copy(rows1, acc_sp.at[idx_d.at[1]], add=True)

            return carry

        lax.fori_loop(0, (nfull + 1) // 2, pair, 0)

    if tail > 0:
        pltpu.sync_copy(dst_hbm.at[pl.ds(base + nfull * CHUNK, tail)], idx_t)
        pltpu.async_copy(
            g_hbm.at[idx_s.at[pl.ds(nfull * CHUNK, tail)]],
            rows0.at[pl.ds(0, tail)], sem0).wait()
        pltpu.sync_copy(rows0.at[pl.ds(0, tail)], acc_sp.at[idx_t], add=True)

    plsc.subcore_barrier()

    def obody(k_, carry):
        r0 = s * rows_per + k_ * CHUNK
        pltpu.sync_copy(acc_sp.at[pl.ds(r0, CHUNK)], rows0)
        pltpu.sync_copy(rows0, out_hbm.at[pl.ds(c * n_pad + r0, CHUNK)])
        return carry

    lax.fori_loop(0, rows_per // CHUNK, obody, 0)


def _aggregate(g, src, dst, n_pad):
    d = g.shape[1]
    ept = src.shape[0] // NW
    tail = ept % CHUNK
    k = pl.kernel(
        functools.partial(_k3_body, n_pad, ept, d),
        out_type=jax.ShapeDtypeStruct((NC * n_pad, d), jnp.float32),
        mesh=_sc_mesh(),
        scratch_types=[
            pltpu.VMEM((ept,), jnp.int32),
            pltpu.VMEM((2, CHUNK), jnp.int32),
            pltpu.VMEM((max(tail, 8),), jnp.int32),
            pltpu.VMEM((CHUNK, d), jnp.float32),
            pltpu.VMEM((CHUNK, d), jnp.float32),
            pltpu.VMEM_SHARED((n_pad, d), jnp.float32),
            pltpu.SemaphoreType.DMA,
            pltpu.SemaphoreType.DMA,
            pltpu.SemaphoreType.DMA,
            pltpu.SemaphoreType.DMA,
        ],
    )
    return k(g, src, dst)


def _k4_body(x_ref, a0_ref, a1_ref, g_ref, d0_ref, d1_ref, b_ref, o_ref):
    dis = lax.rsqrt(d0_ref[...] + d1_ref[...] + 1.0)
    t = dis * (a0_ref[...] + a1_ref[...] + g_ref[...]) + b_ref[...]
    o_ref[...] = x_ref[...] + jnp.maximum(t, 0.0)


def _epilogue(x, a0, a1, g, deg0, deg1, b2d):
    n, d = x.shape
    blk = 2000
    grid = (n // blk,)
    row_spec = pl.BlockSpec((blk, d), lambda i: (i, 0))
    col_spec = pl.BlockSpec((blk, 1), lambda i: (i, 0))
    return pl.pallas_call(
        _k4_body,
        grid=grid,
        in_specs=[row_spec, row_spec, row_spec, row_spec, col_spec, col_spec,
                  pl.BlockSpec((1, d), lambda i: (0, 0))],
        out_specs=row_spec,
        out_shape=jax.ShapeDtypeStruct((n, d), jnp.float32),
    )(x, a0, a1, g, deg0, deg1, b2d)


def kernel(x, edge_index, W, b):
    n, d_in = x.shape
    d_out = W.shape[1]
    e = edge_index.shape[1]

    n_pad = ((n + (NS * CHUNK) - 1) // (NS * CHUNK)) * (NS * CHUNK)

    src = edge_index[0].astype(jnp.int32)
    dst = edge_index[1].astype(jnp.int32)
    qe = NW * 8
    e_pad = ((e + qe - 1) // qe) * qe
    if e_pad != e:
        pad_s = jnp.zeros((e_pad - e,), jnp.int32)
        pad_d = jnp.full((e_pad - e,), n, jnp.int32)
        src = jnp.concatenate([src, pad_s])
        dst = jnp.concatenate([dst, pad_d])

    degp = _deg_partials(dst, n_pad)
    deg0 = degp[:n].reshape(n, 1)
    deg1 = degp[n_pad:n_pad + n].reshape(n, 1)

    g = _scaled_transform(x, W, deg0, deg1)

    acc = _aggregate(g, src, dst, n_pad)

    out = _epilogue(
        x, acc[:n], acc[n_pad:n_pad + n], g,
        deg0, deg1, b.reshape(1, d_out)
    )
    return out


# speedup vs baseline: 38.0014x; 1.1012x over previous
import functools

import jax
import jax.numpy as jnp
from jax import lax
from jax.experimental import pallas as pl
from jax.experimental.pallas import tpu as pltpu
from jax.experimental.pallas import tpu_sc as plsc

NC = 2
NS = 16
NW = NC * NS
CHUNK = 128


def _sc_mesh():
    return plsc.VectorSubcoreMesh(
        core_axis_name="c", subcore_axis_name="s", num_cores=NC, num_subcores=NS
    )


def _k1_body(n_pad, ept, e_off, edge_hbm, out_hbm, idx_d, idx_t, ones_v, tmp_v, deg_sp,
             semd0, semd1):
    c = lax.axis_index("c")
    s = lax.axis_index("s")
    base = (c * NS + s) * ept
    rows_per = n_pad // NS
    nfull = ept // CHUNK
    tail = ept % CHUNK

    for j in range(CHUNK // 16):
        ones_v[pl.ds(j * 16, 16)] = jnp.ones((16,), jnp.float32)

    def zbody(j, carry):
        tmp_v[pl.ds(j * 16, 16)] = jnp.zeros((16,), jnp.float32)
        return carry

    lax.fori_loop(0, rows_per // 16, zbody, 0)
    pltpu.sync_copy(tmp_v, deg_sp.at[pl.ds(s * rows_per, rows_per)])
    plsc.subcore_barrier()

    def dload(j, k, sem):
        pltpu.async_copy(edge_hbm.at[pl.ds(e_off + base + j * CHUNK, CHUNK)],
                         idx_d.at[k], sem)

    def dwait(k, sem):
        pltpu.make_async_copy(edge_hbm.at[pl.ds(0, CHUNK)], idx_d.at[k],
                              sem).wait()

    if nfull > 0:
        dload(0, 0, semd0)

        def pair(p, carry):
            j0 = p * 2
            j1 = j0 + 1

            @pl.when(j1 < nfull)
            def _():
                dload(j1, 1, semd1)

            dwait(0, semd0)
            pltpu.sync_copy(ones_v, deg_sp.at[idx_d.at[0]], add=True)

            @pl.when(j1 < nfull)
            def _():
                @pl.when(j1 + 1 < nfull)
                def _():
                    dload(j1 + 1, 0, semd0)

                dwait(1, semd1)
                pltpu.sync_copy(ones_v, deg_sp.at[idx_d.at[1]], add=True)

            return carry

        lax.fori_loop(0, (nfull + 1) // 2, pair, 0)

    if tail > 0:
        pltpu.sync_copy(edge_hbm.at[pl.ds(e_off + base + nfull * CHUNK, tail)], idx_t)
        pltpu.sync_copy(ones_v.at[pl.ds(0, tail)], deg_sp.at[idx_t], add=True)

    plsc.subcore_barrier()
    pltpu.sync_copy(deg_sp.at[pl.ds(s * rows_per, rows_per)], tmp_v)
    pltpu.sync_copy(tmp_v, out_hbm.at[pl.ds(c * n_pad + s * rows_per, rows_per)])


def _deg_partials(edge, n_pad):
    e_off = edge.shape[0] // 2
    ept = e_off // NW
    tail = ept % CHUNK
    rows_per = n_pad // NS
    k = pl.kernel(
        functools.partial(_k1_body, n_pad, ept, e_off),
        out_type=jax.ShapeDtypeStruct((NC * n_pad,), jnp.float32),
        mesh=_sc_mesh(),
        scratch_types=[
            pltpu.VMEM((2, CHUNK), jnp.int32),
            pltpu.VMEM((max(tail, 8),), jnp.int32),
            pltpu.VMEM((CHUNK,), jnp.float32),
            pltpu.VMEM((rows_per,), jnp.float32),
            pltpu.VMEM_SHARED((n_pad,), jnp.float32),
            pltpu.SemaphoreType.DMA,
            pltpu.SemaphoreType.DMA,
        ],
    )
    return k(edge)


def _k2_body(x_ref, w_ref, d0_ref, d1_ref, g_ref):
    deg = d0_ref[...] + d1_ref[...] + 1.0
    dis = lax.rsqrt(deg)
    h = jnp.dot(x_ref[...], w_ref[...], preferred_element_type=jnp.float32)
    g_ref[...] = h * dis


def _scaled_transform(x, w, deg0, deg1):
    n, d_in = x.shape
    d_out = w.shape[1]
    blk = 2000
    grid = (n // blk,)
    return pl.pallas_call(
        _k2_body,
        grid=grid,
        in_specs=[
            pl.BlockSpec((blk, d_in), lambda i: (i, 0)),
            pl.BlockSpec((d_in, d_out), lambda i: (0, 0)),
            pl.BlockSpec((blk, 1), lambda i: (i, 0)),
            pl.BlockSpec((blk, 1), lambda i: (i, 0)),
        ],
        out_specs=pl.BlockSpec((blk, d_out), lambda i: (i, 0)),
        out_shape=jax.ShapeDtypeStruct((n, d_out), jnp.float32),
    )(x, w, deg0, deg1)


def _k3_body(n_pad, ept, d, e_off, g_hbm, edge_hbm, out_hbm,
             idx_s, idx_d, idx_t, rows0, rows1, acc_sp,
             sem0, sem1, semd0, semd1):
    c = lax.axis_index("c")
    s = lax.axis_index("s")
    base = (c * NS + s) * ept
    rows_per = n_pad // NS
    nfull = ept // CHUNK
    tail = ept % CHUNK

    pltpu.sync_copy(edge_hbm.at[pl.ds(base, ept)], idx_s)

    def zrow(r, carry):
        def zcol(j, carry2):
            rows0[r, pl.ds(j * 16, 16)] = jnp.zeros((16,), jnp.float32)
            return carry2
        return lax.fori_loop(0, d // 16, zcol, carry)

    lax.fori_loop(0, CHUNK, zrow, 0)

    def ibody(k_, carry):
        pltpu.sync_copy(rows0, acc_sp.at[pl.ds(s * rows_per + k_ * CHUNK, CHUNK)])
        return carry

    lax.fori_loop(0, rows_per // CHUNK, ibody, 0)
    plsc.subcore_barrier()

    def gather(j, rows, sem):
        pltpu.async_copy(g_hbm.at[idx_s.at[pl.ds(j * CHUNK, CHUNK)]], rows, sem)

    def gwait(rows, sem):
        pltpu.make_async_copy(g_hbm.at[idx_s.at[pl.ds(0, CHUNK)]], rows,
                              sem).wait()

    def dload(j, k, sem):
        pltpu.async_copy(edge_hbm.at[pl.ds(e_off + base + j * CHUNK, CHUNK)],
                         idx_d.at[k], sem)

    def dwait(k, sem):
        pltpu.make_async_copy(edge_hbm.at[pl.ds(0, CHUNK)], idx_d.at[k],
                              sem).wait()

    if nfull > 0:
        dload(0, 0, semd0)
        gather(0, rows0, sem0)

        def pair(p, carry):
            j0 = p * 2
            j1 = j0 + 1

            @pl.when(j1 < nfull)
            def _():
                dload(j1, 1, semd1)
                gather(j1, rows1, sem1)

            gwait(rows0, sem0)
            dwait(0, semd0)
            pltpu.sync_copy(rows0, acc_sp.at[idx_d.at[0]], add=True)

            @pl.when(j1 < nfull)
            def _():
                @pl.when(j1 + 1 < nfull)
                def _():
                    dload(j1 + 1, 0, semd0)
                    gather(j1 + 1, rows0, sem0)

                gwait(rows1, sem1)
                dwait(1, semd1)
                pltpu.sync_copy(rows1, acc_sp.at[idx_d.at[1]], add=True)

            return carry

        lax.fori_loop(0, (nfull + 1) // 2, pair, 0)

    if tail > 0:
        pltpu.sync_copy(edge_hbm.at[pl.ds(e_off + base + nfull * CHUNK, tail)], idx_t)
        pltpu.async_copy(
            g_hbm.at[idx_s.at[pl.ds(nfull * CHUNK, tail)]],
            rows0.at[pl.ds(0, tail)], sem0).wait()
        pltpu.sync_copy(rows0.at[pl.ds(0, tail)], acc_sp.at[idx_t], add=True)

    plsc.subcore_barrier()

    def obody(k_, carry):
        r0 = s * rows_per + k_ * CHUNK
        pltpu.sync_copy(acc_sp.at[pl.ds(r0, CHUNK)], rows0)
        pltpu.sync_copy(rows0, out_hbm.at[pl.ds(c * n_pad + r0, CHUNK)])
        return carry

    lax.fori_loop(0, rows_per // CHUNK, obody, 0)


def _aggregate(g, edge, n_pad):
    d = g.shape[1]
    e_off = edge.shape[0] // 2
    ept = e_off // NW
    tail = ept % CHUNK
    k = pl.kernel(
        functools.partial(_k3_body, n_pad, ept, d, e_off),
        out_type=jax.ShapeDtypeStruct((NC * n_pad, d), jnp.float32),
        mesh=_sc_mesh(),
        scratch_types=[
            pltpu.VMEM((ept,), jnp.int32),
            pltpu.VMEM((2, CHUNK), jnp.int32),
            pltpu.VMEM((max(tail, 8),), jnp.int32),
            pltpu.VMEM((CHUNK, d), jnp.float32),
            pltpu.VMEM((CHUNK, d), jnp.float32),
            pltpu.VMEM_SHARED((n_pad, d), jnp.float32),
            pltpu.SemaphoreType.DMA,
            pltpu.SemaphoreType.DMA,
            pltpu.SemaphoreType.DMA,
            pltpu.SemaphoreType.DMA,
        ],
    )
    return k(g, edge)


def _k4_body(x_ref, a0_ref, a1_ref, g_ref, d0_ref, d1_ref, b_ref, o_ref):
    dis = lax.rsqrt(d0_ref[...] + d1_ref[...] + 1.0)
    t = dis * (a0_ref[0] + a1_ref[0] + g_ref[...]) + b_ref[...]
    o_ref[...] = x_ref[...] + jnp.maximum(t, 0.0)


def _epilogue(x, acc, g, deg0, deg1, b2d):
    n, d = x.shape
    blk = 2000
    grid = (n // blk,)
    row_spec = pl.BlockSpec((blk, d), lambda i: (i, 0))
    col_spec = pl.BlockSpec((blk, 1), lambda i: (i, 0))
    return pl.pallas_call(
        _k4_body,
        grid=grid,
        in_specs=[row_spec,
                  pl.BlockSpec((1, blk, d), lambda i: (0, i, 0)),
                  pl.BlockSpec((1, blk, d), lambda i: (1, i, 0)),
                  row_spec, col_spec, col_spec,
                  pl.BlockSpec((1, d), lambda i: (0, 0))],
        out_specs=row_spec,
        out_shape=jax.ShapeDtypeStruct((n, d), jnp.float32),
    )(x, acc, acc, g, deg0, deg1, b2d)


def kernel(x, edge_index, W, b):
    n, d_in = x.shape
    d_out = W.shape[1]
    e = edge_index.shape[1]

    n_pad = ((n + (NS * CHUNK) - 1) // (NS * CHUNK)) * (NS * CHUNK)

    edge = edge_index.astype(jnp.int32)
    qe = NW * 8
    e_pad = ((e + qe - 1) // qe) * qe
    if e_pad != e:
        pad = jnp.stack([jnp.zeros((e_pad - e,), jnp.int32),
                         jnp.full((e_pad - e,), n, jnp.int32)])
        edge = jnp.concatenate([edge, pad], axis=1)
    edge = edge.reshape(2 * e_pad)

    degp = _deg_partials(edge, n_pad)
    deg0 = degp[:n].reshape(n, 1)
    deg1 = degp[n_pad:n_pad + n].reshape(n, 1)

    g = _scaled_transform(x, W, deg0, deg1)

    acc = _aggregate(g, edge, n_pad)
    acc3 = acc.reshape(NC, n_pad, d_out)

    out = _epilogue(x, acc3, g, deg0, deg1, b.reshape(1, d_out))
    return out


# speedup vs baseline: 38.0170x; 1.0004x over previous
import functools

import jax
import jax.numpy as jnp
from jax import lax
from jax.experimental import pallas as pl
from jax.experimental.pallas import tpu as pltpu
from jax.experimental.pallas import tpu_sc as plsc

NC = 2
NS = 16
NW = NC * NS
CHUNK = 128


def _sc_mesh():
    return plsc.VectorSubcoreMesh(
        core_axis_name="c", subcore_axis_name="s", num_cores=NC, num_subcores=NS
    )


def _k1_body(n_pad, ept, e_off, edge_hbm, out_hbm, idx_d, idx_t, ones_v, tmp_v, deg_sp,
             semd0, semd1, sems0, sems1):
    c = lax.axis_index("c")
    s = lax.axis_index("s")
    base = (c * NS + s) * ept
    rows_per = n_pad // NS
    nfull = ept // CHUNK
    tail = ept % CHUNK

    for j in range(CHUNK // 16):
        ones_v[pl.ds(j * 16, 16)] = jnp.ones((16,), jnp.float32)

    def zbody(j, carry):
        tmp_v[pl.ds(j * 16, 16)] = jnp.zeros((16,), jnp.float32)
        return carry

    lax.fori_loop(0, rows_per // 16, zbody, 0)
    pltpu.sync_copy(tmp_v, deg_sp.at[pl.ds(s * rows_per, rows_per)])
    plsc.subcore_barrier()

    def dload(j, k, sem):
        pltpu.async_copy(edge_hbm.at[pl.ds(e_off + base + j * CHUNK, CHUNK)],
                         idx_d.at[k], sem)

    def dwait(k, sem):
        pltpu.make_async_copy(edge_hbm.at[pl.ds(0, CHUNK)], idx_d.at[k],
                              sem).wait()

    def scat(k, sem):
        pltpu.async_copy(ones_v, deg_sp.at[idx_d.at[k]], sem, add=True)

    def swait(k, sem):
        pltpu.make_async_copy(ones_v, deg_sp.at[idx_d.at[k]], sem).wait()

    if nfull > 0:
        dload(0, 0, semd0)

        def pair(p, carry):
            j0 = p * 2
            j1 = j0 + 1

            @pl.when(j1 < nfull)
            def _():
                @pl.when(p > 0)
                def _():
                    swait(1, sems1)

                dload(j1, 1, semd1)

            dwait(0, semd0)
            scat(0, sems0)

            @pl.when(j1 < nfull)
            def _():
                @pl.when(j1 + 1 < nfull)
                def _():
                    swait(0, sems0)
                    dload(j1 + 1, 0, semd0)

                dwait(1, semd1)
                scat(1, sems1)

            return carry

        lax.fori_loop(0, (nfull + 1) // 2, pair, 0)
        swait(0, sems0)
        if nfull > 1:
            swait(1, sems1)

    if tail > 0:
        pltpu.sync_copy(edge_hbm.at[pl.ds(e_off + base + nfull * CHUNK, tail)], idx_t)
        pltpu.sync_copy(ones_v.at[pl.ds(0, tail)], deg_sp.at[idx_t], add=True)

    plsc.subcore_barrier()
    pltpu.sync_copy(deg_sp.at[pl.ds(s * rows_per, rows_per)], tmp_v)
    pltpu.sync_copy(tmp_v, out_hbm.at[pl.ds(c * n_pad + s * rows_per, rows_per)])


def _deg_partials(edge, n_pad):
    e_off = edge.shape[0] // 2
    ept = e_off // NW
    tail = ept % CHUNK
    rows_per = n_pad // NS
    k = pl.kernel(
        functools.partial(_k1_body, n_pad, ept, e_off),
        out_type=jax.ShapeDtypeStruct((NC * n_pad,), jnp.float32),
        mesh=_sc_mesh(),
        scratch_types=[
            pltpu.VMEM((2, CHUNK), jnp.int32),
            pltpu.VMEM((max(tail, 8),), jnp.int32),
            pltpu.VMEM((CHUNK,), jnp.float32),
            pltpu.VMEM((rows_per,), jnp.float32),
            pltpu.VMEM_SHARED((n_pad,), jnp.float32),
            pltpu.SemaphoreType.DMA,
            pltpu.SemaphoreType.DMA,
            pltpu.SemaphoreType.DMA,
            pltpu.SemaphoreType.DMA,
        ],
    )
    return k(edge)


def _k2a_body(x_ref, w_ref, h_ref):
    h_ref[...] = jnp.dot(x_ref[...], w_ref[...],
                         preferred_element_type=jnp.float32)


def _transform(x, w):
    n, d_in = x.shape
    d_out = w.shape[1]
    blk = 2000
    return pl.pallas_call(
        _k2a_body,
        grid=(n // blk,),
        in_specs=[
            pl.BlockSpec((blk, d_in), lambda i: (i, 0)),
            pl.BlockSpec((d_in, d_out), lambda i: (0, 0)),
        ],
        out_specs=pl.BlockSpec((blk, d_out), lambda i: (i, 0)),
        out_shape=jax.ShapeDtypeStruct((n, d_out), jnp.float32),
    )(x, w)


def _k2b_body(h_ref, d0_ref, d1_ref, g_ref):
    dis = lax.rsqrt(d0_ref[...] + d1_ref[...] + 1.0)
    g_ref[...] = h_ref[...] * dis


def _scale(h, deg0, deg1):
    n, d = h.shape
    blk = 2000
    row_spec = pl.BlockSpec((blk, d), lambda i: (i, 0))
    col_spec = pl.BlockSpec((blk, 1), lambda i: (i, 0))
    return pl.pallas_call(
        _k2b_body,
        grid=(n // blk,),
        in_specs=[row_spec, col_spec, col_spec],
        out_specs=row_spec,
        out_shape=jax.ShapeDtypeStruct((n, d), jnp.float32),
    )(h, deg0, deg1)


def _k3_body(n_pad, ept, d, e_off, g_hbm, edge_hbm, out_hbm,
             idx_s, idx_d, idx_t, rows0, rows1, acc_sp,
             sem0, sem1, semd0, semd1):
    c = lax.axis_index("c")
    s = lax.axis_index("s")
    base = (c * NS + s) * ept
    rows_per = n_pad // NS
    nfull = ept // CHUNK
    tail = ept % CHUNK

    pltpu.sync_copy(edge_hbm.at[pl.ds(base, ept)], idx_s)

    def zrow(r, carry):
        def zcol(j, carry2):
            rows0[r, pl.ds(j * 16, 16)] = jnp.zeros((16,), jnp.float32)
            return carry2
        return lax.fori_loop(0, d // 16, zcol, carry)

    lax.fori_loop(0, CHUNK, zrow, 0)

    def ibody(k_, carry):
        pltpu.sync_copy(rows0, acc_sp.at[pl.ds(s * rows_per + k_ * CHUNK, CHUNK)])
        return carry

    lax.fori_loop(0, rows_per // CHUNK, ibody, 0)
    plsc.subcore_barrier()

    def gather(j, rows, sem):
        pltpu.async_copy(g_hbm.at[idx_s.at[pl.ds(j * CHUNK, CHUNK)]], rows, sem)

    def gwait(rows, sem):
        pltpu.make_async_copy(g_hbm.at[idx_s.at[pl.ds(0, CHUNK)]], rows,
                              sem).wait()

    def dload(j, k, sem):
        pltpu.async_copy(edge_hbm.at[pl.ds(e_off + base + j * CHUNK, CHUNK)],
                         idx_d.at[k], sem)

    def dwait(k, sem):
        pltpu.make_async_copy(edge_hbm.at[pl.ds(0, CHUNK)], idx_d.at[k],
                              sem).wait()

    if nfull > 0:
        dload(0, 0, semd0)
        gather(0, rows0, sem0)

        def pair(p, carry):
            j0 = p * 2
            j1 = j0 + 1

            @pl.when(j1 < nfull)
            def _():
                dload(j1, 1, semd1)
                gather(j1, rows1, sem1)

            gwait(rows0, sem0)
            dwait(0, semd0)
            pltpu.sync_copy(rows0, acc_sp.at[idx_d.at[0]], add=True)

            @pl.when(j1 < nfull)
            def _():
                @pl.when(j1 + 1 < nfull)
                def _():
                    dload(j1 + 1, 0, semd0)
                    gather(j1 + 1, rows0, sem0)

                gwait(rows1, sem1)
                dwait(1, semd1)
                pltpu.sync_copy(rows1, acc_sp.at[idx_d.at[1]], add=True)

            return carry

        lax.fori_loop(0, (nfull + 1) // 2, pair, 0)

    if tail > 0:
        pltpu.sync_copy(edge_hbm.at[pl.ds(e_off + base + nfull * CHUNK, tail)], idx_t)
        pltpu.async_copy(
            g_hbm.at[idx_s.at[pl.ds(nfull * CHUNK, tail)]],
            rows0.at[pl.ds(0, tail)], sem0).wait()
        pltpu.sync_copy(rows0.at[pl.ds(0, tail)], acc_sp.at[idx_t], add=True)

    plsc.subcore_barrier()

    nout = rows_per // CHUNK

    def oread(k_, rows, sem):
        pltpu.async_copy(acc_sp.at[pl.ds(s * rows_per + k_ * CHUNK, CHUNK)],
                         rows, sem)

    def owait(rows, sem):
        pltpu.make_async_copy(acc_sp.at[pl.ds(0, CHUNK)], rows, sem).wait()

    oread(0, rows0, sem0)

    def opair(p, carry):
        k0 = p * 2
        k1 = k0 + 1

        @pl.when(k1 < nout)
        def _():
            oread(k1, rows1, sem1)

        owait(rows0, sem0)
        pltpu.sync_copy(rows0, out_hbm.at[pl.ds(c * n_pad + k0 * CHUNK
                                                + s * rows_per, CHUNK)])

        @pl.when(k1 < nout)
        def _():
            @pl.when(k1 + 1 < nout)
            def _():
                oread(k1 + 1, rows0, sem0)

            owait(rows1, sem1)
            pltpu.sync_copy(rows1, out_hbm.at[pl.ds(c * n_pad + k1 * CHUNK
                                                    + s * rows_per, CHUNK)])

        return carry

    lax.fori_loop(0, (nout + 1) // 2, opair, 0)


def _aggregate(g, edge, n_pad):
    d = g.shape[1]
    e_off = edge.shape[0] // 2
    ept = e_off // NW
    tail = ept % CHUNK
    k = pl.kernel(
        functools.partial(_k3_body, n_pad, ept, d, e_off),
        out_type=jax.ShapeDtypeStruct((NC * n_pad, d), jnp.float32),
        mesh=_sc_mesh(),
        scratch_types=[
            pltpu.VMEM((ept,), jnp.int32),
            pltpu.VMEM((2, CHUNK), jnp.int32),
            pltpu.VMEM((max(tail, 8),), jnp.int32),
            pltpu.VMEM((CHUNK, d), jnp.float32),
            pltpu.VMEM((CHUNK, d), jnp.float32),
            pltpu.VMEM_SHARED((n_pad, d), jnp.float32),
            pltpu.SemaphoreType.DMA,
            pltpu.SemaphoreType.DMA,
            pltpu.SemaphoreType.DMA,
            pltpu.SemaphoreType.DMA,
        ],
    )
    return k(g, edge)


def _k4_body(x_ref, a0_ref, a1_ref, g_ref, d0_ref, d1_ref, b_ref, o_ref):
    dis = lax.rsqrt(d0_ref[...] + d1_ref[...] + 1.0)
    t = dis * (a0_ref[0] + a1_ref[0] + g_ref[...]) + b_ref[...]
    o_ref[...] = x_ref[...] + jnp.maximum(t, 0.0)


def _epilogue(x, acc, g, deg0, deg1, b2d):
    n, d = x.shape
    blk = 2000
    grid = (n // blk,)
    row_spec = pl.BlockSpec((blk, d), lambda i: (i, 0))
    col_spec = pl.BlockSpec((blk, 1), lambda i: (i, 0))
    return pl.pallas_call(
        _k4_body,
        grid=grid,
        in_specs=[row_spec,
                  pl.BlockSpec((1, blk, d), lambda i: (0, i, 0)),
                  pl.BlockSpec((1, blk, d), lambda i: (1, i, 0)),
                  row_spec, col_spec, col_spec,
                  pl.BlockSpec((1, d), lambda i: (0, 0))],
        out_specs=row_spec,
        out_shape=jax.ShapeDtypeStruct((n, d), jnp.float32),
    )(x, acc, acc, g, deg0, deg1, b2d)


def kernel(x, edge_index, W, b):
    n, d_in = x.shape
    d_out = W.shape[1]
    e = edge_index.shape[1]

    n_pad = ((n + (NS * CHUNK) - 1) // (NS * CHUNK)) * (NS * CHUNK)

    edge = edge_index.astype(jnp.int32)
    qe = NW * 8
    e_pad = ((e + qe - 1) // qe) * qe
    if e_pad != e:
        pad = jnp.stack([jnp.zeros((e_pad - e,), jnp.int32),
                         jnp.full((e_pad - e,), n, jnp.int32)])
        edge = jnp.concatenate([edge, pad], axis=1)
    edge = edge.reshape(2 * e_pad)

    degp = _deg_partials(edge, n_pad)
    h = _transform(x, W)
    deg0 = degp[:n].reshape(n, 1)
    deg1 = degp[n_pad:n_pad + n].reshape(n, 1)

    g = _scale(h, deg0, deg1)

    acc = _aggregate(g, edge, n_pad)
    acc3 = acc.reshape(NC, n_pad, d_out)

    out = _epilogue(x, acc3, g, deg0, deg1, b.reshape(1, d_out))
    return out


# speedup vs baseline: 39.7175x; 1.0447x over previous
import functools

import jax
import jax.numpy as jnp
from jax import lax
from jax.experimental import pallas as pl
from jax.experimental.pallas import tpu as pltpu
from jax.experimental.pallas import tpu_sc as plsc

NC = 2
NS = 16
NW = NC * NS
CHUNK = 128


def _sc_mesh():
    return plsc.VectorSubcoreMesh(
        core_axis_name="c", subcore_axis_name="s", num_cores=NC, num_subcores=NS
    )


def _k1_body(n_pad, ept, e_off, edge_hbm, out_hbm, idx_d, idx_t, ones_v, tmp_v, deg_sp,
             semd0, semd1, sems0, sems1):
    c = lax.axis_index("c")
    s = lax.axis_index("s")
    base = (c * NS + s) * ept
    rows_per = n_pad // NS
    nfull = ept // CHUNK
    tail = ept % CHUNK

    for j in range(CHUNK // 16):
        ones_v[pl.ds(j * 16, 16)] = jnp.ones((16,), jnp.float32)

    def zbody(j, carry):
        tmp_v[pl.ds(j * 16, 16)] = jnp.zeros((16,), jnp.float32)
        return carry

    lax.fori_loop(0, rows_per // 16, zbody, 0)
    pltpu.sync_copy(tmp_v, deg_sp.at[pl.ds(s * rows_per, rows_per)])
    plsc.subcore_barrier()

    def dload(j, k, sem):
        pltpu.async_copy(edge_hbm.at[pl.ds(e_off + base + j * CHUNK, CHUNK)],
                         idx_d.at[k], sem)

    def dwait(k, sem):
        pltpu.make_async_copy(edge_hbm.at[pl.ds(0, CHUNK)], idx_d.at[k],
                              sem).wait()

    def scat(k, sem):
        pltpu.async_copy(ones_v, deg_sp.at[idx_d.at[k]], sem, add=True)

    def swait(k, sem):
        pltpu.make_async_copy(ones_v, deg_sp.at[idx_d.at[k]], sem).wait()

    if nfull > 0:
        dload(0, 0, semd0)

        def pair(p, carry):
            j0 = p * 2
            j1 = j0 + 1

            @pl.when(j1 < nfull)
            def _():
                @pl.when(p > 0)
                def _():
                    swait(1, sems1)

                dload(j1, 1, semd1)

            dwait(0, semd0)
            scat(0, sems0)

            @pl.when(j1 < nfull)
            def _():
                @pl.when(j1 + 1 < nfull)
                def _():
                    swait(0, sems0)
                    dload(j1 + 1, 0, semd0)

                dwait(1, semd1)
                scat(1, sems1)

            return carry

        lax.fori_loop(0, (nfull + 1) // 2, pair, 0)
        swait(0, sems0)
        if nfull > 1:
            swait(1, sems1)

    if tail > 0:
        pltpu.sync_copy(edge_hbm.at[pl.ds(e_off + base + nfull * CHUNK, tail)], idx_t)
        pltpu.sync_copy(ones_v.at[pl.ds(0, tail)], deg_sp.at[idx_t], add=True)

    plsc.subcore_barrier()
    pltpu.sync_copy(deg_sp.at[pl.ds(s * rows_per, rows_per)], tmp_v)
    pltpu.sync_copy(tmp_v, out_hbm.at[pl.ds(c * n_pad + s * rows_per, rows_per)])


def _deg_partials(edge, n_pad):
    e_off = edge.shape[0] // 2
    ept = e_off // NW
    tail = ept % CHUNK
    rows_per = n_pad // NS
    k = pl.kernel(
        functools.partial(_k1_body, n_pad, ept, e_off),
        out_type=jax.ShapeDtypeStruct((NC * n_pad,), jnp.float32),
        mesh=_sc_mesh(),
        scratch_types=[
            pltpu.VMEM((2, CHUNK), jnp.int32),
            pltpu.VMEM((max(tail, 8),), jnp.int32),
            pltpu.VMEM((CHUNK,), jnp.float32),
            pltpu.VMEM((rows_per,), jnp.float32),
            pltpu.VMEM_SHARED((n_pad,), jnp.float32),
            pltpu.SemaphoreType.DMA,
            pltpu.SemaphoreType.DMA,
            pltpu.SemaphoreType.DMA,
            pltpu.SemaphoreType.DMA,
        ],
    )
    return k(edge)


def _k2a_body(x_ref, w_ref, h_ref):
    h_ref[...] = jnp.dot(x_ref[...], w_ref[...],
                         preferred_element_type=jnp.float32)


def _transform(x, w):
    n, d_in = x.shape
    d_out = w.shape[1]
    blk = 2000
    return pl.pallas_call(
        _k2a_body,
        grid=(n // blk,),
        in_specs=[
            pl.BlockSpec((blk, d_in), lambda i: (i, 0)),
            pl.BlockSpec((d_in, d_out), lambda i: (0, 0)),
        ],
        out_specs=pl.BlockSpec((blk, d_out), lambda i: (i, 0)),
        out_shape=jax.ShapeDtypeStruct((n, d_out), jnp.float32),
    )(x, w)


def _k2b_body(h_ref, dsum_ref, g_ref):
    dis = lax.rsqrt(dsum_ref[...] + 1.0)
    g_ref[...] = h_ref[...] * dis


def _scale(h, degsum):
    n, d = h.shape
    blk = 2000
    row_spec = pl.BlockSpec((blk, d), lambda i: (i, 0))
    col_spec = pl.BlockSpec((blk, 1), lambda i: (i, 0))
    return pl.pallas_call(
        _k2b_body,
        grid=(n // blk,),
        in_specs=[row_spec, col_spec],
        out_specs=row_spec,
        out_shape=jax.ShapeDtypeStruct((n, d), jnp.float32),
    )(h, degsum)


def _k3_body(n_pad, ept, d, e_off, g_hbm, edge_hbm, out_hbm,
             idx_s, idx_d, idx_t, rows0, rows1, acc_sp,
             sem0, sem1, semd0, semd1):
    c = lax.axis_index("c")
    s = lax.axis_index("s")
    base = (c * NS + s) * ept
    rows_per = n_pad // NS
    nfull = ept // CHUNK
    tail = ept % CHUNK

    pltpu.sync_copy(edge_hbm.at[pl.ds(base, ept)], idx_s)

    def zrow(r, carry):
        def zcol(j, carry2):
            rows0[r, pl.ds(j * 16, 16)] = jnp.zeros((16,), jnp.float32)
            return carry2
        return lax.fori_loop(0, d // 16, zcol, carry)

    lax.fori_loop(0, CHUNK, zrow, 0)

    def ibody(k_, carry):
        pltpu.sync_copy(rows0, acc_sp.at[pl.ds(s * rows_per + k_ * CHUNK, CHUNK)])
        return carry

    lax.fori_loop(0, rows_per // CHUNK, ibody, 0)
    plsc.subcore_barrier()

    def gather(j, rows, sem):
        pltpu.async_copy(g_hbm.at[idx_s.at[pl.ds(j * CHUNK, CHUNK)]], rows, sem)

    def gwait(rows, sem):
        pltpu.make_async_copy(g_hbm.at[idx_s.at[pl.ds(0, CHUNK)]], rows,
                              sem).wait()

    def dload(j, k, sem):
        pltpu.async_copy(edge_hbm.at[pl.ds(e_off + base + j * CHUNK, CHUNK)],
                         idx_d.at[k], sem)

    def dwait(k, sem):
        pltpu.make_async_copy(edge_hbm.at[pl.ds(0, CHUNK)], idx_d.at[k],
                              sem).wait()

    if nfull > 0:
        dload(0, 0, semd0)
        gather(0, rows0, sem0)

        def pair(p, carry):
            j0 = p * 2
            j1 = j0 + 1

            @pl.when(j1 < nfull)
            def _():
                dload(j1, 1, semd1)
                gather(j1, rows1, sem1)

            gwait(rows0, sem0)
            dwait(0, semd0)
            pltpu.sync_copy(rows0, acc_sp.at[idx_d.at[0]], add=True)

            @pl.when(j1 < nfull)
            def _():
                @pl.when(j1 + 1 < nfull)
                def _():
                    dload(j1 + 1, 0, semd0)
                    gather(j1 + 1, rows0, sem0)

                gwait(rows1, sem1)
                dwait(1, semd1)
                pltpu.sync_copy(rows1, acc_sp.at[idx_d.at[1]], add=True)

            return carry

        lax.fori_loop(0, (nfull + 1) // 2, pair, 0)

    if tail > 0:
        pltpu.sync_copy(edge_hbm.at[pl.ds(e_off + base + nfull * CHUNK, tail)], idx_t)
        pltpu.async_copy(
            g_hbm.at[idx_s.at[pl.ds(nfull * CHUNK, tail)]],
            rows0.at[pl.ds(0, tail)], sem0).wait()
        pltpu.sync_copy(rows0.at[pl.ds(0, tail)], acc_sp.at[idx_t], add=True)

    plsc.subcore_barrier()

    nout = rows_per // CHUNK

    def oread(k_, rows, sem):
        pltpu.async_copy(acc_sp.at[pl.ds(s * rows_per + k_ * CHUNK, CHUNK)],
                         rows, sem)

    def owait(rows, sem):
        pltpu.make_async_copy(acc_sp.at[pl.ds(0, CHUNK)], rows, sem).wait()

    oread(0, rows0, sem0)

    def opair(p, carry):
        k0 = p * 2
        k1 = k0 + 1

        @pl.when(k1 < nout)
        def _():
            oread(k1, rows1, sem1)

        owait(rows0, sem0)
        pltpu.sync_copy(rows0, out_hbm.at[pl.ds(c * n_pad + k0 * CHUNK
                                                + s * rows_per, CHUNK)])

        @pl.when(k1 < nout)
        def _():
            @pl.when(k1 + 1 < nout)
            def _():
                oread(k1 + 1, rows0, sem0)

            owait(rows1, sem1)
            pltpu.sync_copy(rows1, out_hbm.at[pl.ds(c * n_pad + k1 * CHUNK
                                                    + s * rows_per, CHUNK)])

        return carry

    lax.fori_loop(0, (nout + 1) // 2, opair, 0)


def _aggregate(g, edge, n_pad):
    d = g.shape[1]
    e_off = edge.shape[0] // 2
    ept = e_off // NW
    tail = ept % CHUNK
    k = pl.kernel(
        functools.partial(_k3_body, n_pad, ept, d, e_off),
        out_type=jax.ShapeDtypeStruct((NC * n_pad, d), jnp.float32),
        mesh=_sc_mesh(),
        scratch_types=[
            pltpu.VMEM((ept,), jnp.int32),
            pltpu.VMEM((2, CHUNK), jnp.int32),
            pltpu.VMEM((max(tail, 8),), jnp.int32),
            pltpu.VMEM((CHUNK, d), jnp.float32),
            pltpu.VMEM((CHUNK, d), jnp.float32),
            pltpu.VMEM_SHARED((n_pad, d), jnp.float32),
            pltpu.SemaphoreType.DMA,
            pltpu.SemaphoreType.DMA,
            pltpu.SemaphoreType.DMA,
            pltpu.SemaphoreType.DMA,
        ],
    )
    return k(g, edge)


def _k4_body(x_ref, a0_ref, a1_ref, g_ref, dsum_ref, b_ref, o_ref):
    dis = lax.rsqrt(dsum_ref[...] + 1.0)
    t = dis * (a0_ref[0] + a1_ref[0] + g_ref[...]) + b_ref[...]
    o_ref[...] = x_ref[...] + jnp.maximum(t, 0.0)


def _epilogue(x, acc, g, degsum, b2d):
    n, d = x.shape
    blk = 2000
    grid = (n // blk,)
    row_spec = pl.BlockSpec((blk, d), lambda i: (i, 0))
    col_spec = pl.BlockSpec((blk, 1), lambda i: (i, 0))
    return pl.pallas_call(
        _k4_body,
        grid=grid,
        in_specs=[row_spec,
                  pl.BlockSpec((1, blk, d), lambda i: (0, i, 0)),
                  pl.BlockSpec((1, blk, d), lambda i: (1, i, 0)),
                  row_spec, col_spec,
                  pl.BlockSpec((1, d), lambda i: (0, 0))],
        out_specs=row_spec,
        out_shape=jax.ShapeDtypeStruct((n, d), jnp.float32),
    )(x, acc, acc, g, degsum, b2d)


def kernel(x, edge_index, W, b):
    n, d_in = x.shape
    d_out = W.shape[1]
    e = edge_index.shape[1]

    n_pad = ((n + (NS * CHUNK) - 1) // (NS * CHUNK)) * (NS * CHUNK)

    edge = edge_index.astype(jnp.int32)
    qe = NW * 8
    e_pad = ((e + qe - 1) // qe) * qe
    if e_pad != e:
        pad = jnp.stack([jnp.zeros((e_pad - e,), jnp.int32),
                         jnp.full((e_pad - e,), n, jnp.int32)])
        edge = jnp.concatenate([edge, pad], axis=1)
    edge = edge.reshape(2 * e_pad)

    degp = _deg_partials(edge, n_pad)
    h = _transform(x, W)
    degsum = (degp[:n] + degp[n_pad:n_pad + n]).reshape(n, 1)

    g = _scale(h, degsum)

    acc = _aggregate(g, edge, n_pad)
    acc3 = acc.reshape(NC, n_pad, d_out)

    out = _epilogue(x, acc3, g, degsum, b.reshape(1, d_out))
    return out
